# baseline (speedup 1.0000x reference)
import functools

import jax
import jax.numpy as jnp
from jax.experimental import pallas as pl
from jax.experimental.pallas import tpu as pltpu

_ROWS_PER_TILE = 2048


def _mod_kernel(idx_ref, row_ref, W_ref, b_ref, out_ref):
    code = row_ref[0, 0, :]
    mods = jnp.sum(W_ref[...] * code[None, :], axis=1) + b_ref[0, :]
    out_ref[...] = jnp.broadcast_to(mods[None, :], out_ref.shape)


def kernel(coords, patient_idx, emb_table, W, b):
    n = coords.shape[0]
    num_out, code_dim = W.shape
    idx = jnp.asarray(patient_idx, jnp.int32).reshape((1,))
    grid = (n // _ROWS_PER_TILE,)
    out = pl.pallas_call(
        _mod_kernel,
        grid_spec=pltpu.PrefetchScalarGridSpec(
            num_scalar_prefetch=1,
            grid=grid,
            in_specs=[
                pl.BlockSpec((1, 1, code_dim), lambda i, idx_ref: (idx_ref[0], 0, 0)),
                pl.BlockSpec((num_out, code_dim), lambda i, idx_ref: (0, 0)),
                pl.BlockSpec((1, num_out), lambda i, idx_ref: (0, 0)),
            ],
            out_specs=pl.BlockSpec((_ROWS_PER_TILE, num_out), lambda i, idx_ref: (i, 0)),
        ),
        out_shape=jax.ShapeDtypeStruct((n, num_out), jnp.float32),
    )(idx, emb_table.reshape(-1, 1, code_dim), W, b.reshape(1, num_out))
    return out
